# baseline (speedup 1.0000x reference)
import jax
import jax.numpy as jnp
from jax.experimental import pallas as pl
from jax.experimental.pallas import tpu as pltpu

B, P, C = 32, 8732, 21
TP = 384
NP = 23
TPB = 1152
NPB = 8
HALF = B // 2


def _roll(x):
    return jnp.concatenate([x[HALF:], x[:HALF]], axis=0)


def _conf_kernel(lam_ref, conf_ref, confsh_ref, confi_ref, out_ref, ml_ref,
                 mr_ref, acc_ref):
    ip = pl.program_id(0)
    lam = lam_ref[0]
    q = 1.0 - lam

    lanes = jax.lax.broadcasted_iota(jnp.int32, (1, TP), 1)
    valid = (ip * TP + lanes) < P
    validf = valid.astype(jnp.float32)

    tc0 = conf_ref[0]
    tt0 = _roll(confsh_ref[0])

    s_ab = jnp.zeros((B, TP), jnp.float32)
    s_l = jnp.zeros((B, TP), jnp.float32)
    s_r = jnp.zeros((B, TP), jnp.float32)
    m_l = tc0
    m_r = tt0
    for c in range(C):
        tc_c = tc0 if c == 0 else conf_ref[c]
        tt_c = tt0 if c == 0 else _roll(confsh_ref[c])
        if c > 0:
            m_l = jnp.maximum(m_l, tc_c)
            m_r = jnp.maximum(m_r, tt_c)
        interp = confi_ref[c] + 1e-07
        mixed = lam * tc_c + q * tt_c + 1e-07
        log_i = jnp.log(interp)
        d = log_i - jnp.log(mixed)
        s_ab += (interp - mixed) * d
        ce = tc_c + 1e-07
        cte = tt_c + 1e-07
        s_l += ce * (jnp.log(ce) - log_i)
        s_r += cte * (jnp.log(cte) - log_i)

    lmf = (m_l > tc0).astype(jnp.float32) * validf
    rmf = (m_r > tt0).astype(jnp.float32) * validf
    inter = lmf * rmf
    only_l = lmf - inter
    only_r = rmf - inter
    ml_ref[...] = only_l
    mr_ref[...] = only_r

    s_ab = jnp.where(valid, s_ab, 0.0)
    s_l = jnp.where(valid, s_l, 0.0)
    s_r = jnp.where(valid, s_r, 0.0)

    @pl.when(ip == 0)
    def _init():
        acc_ref[...] = jnp.zeros((6, B, TP), jnp.float32)

    acc_ref[0] += s_ab * inter
    acc_ref[1] += inter
    acc_ref[2] += s_l * only_l
    acc_ref[3] += only_l
    acc_ref[4] += s_r * only_r
    acc_ref[5] += only_r

    @pl.when(ip == NP - 1)
    def _fin():
        for j in range(6):
            out_ref[j] = jnp.sum(acc_ref[j])


def _loc_kernel(ml_ref, mr_ref, loc_ref, locsh_ref, loci_ref, out_ref,
                acc_ref):
    ip = pl.program_id(0)

    lanes = jax.lax.broadcasted_iota(jnp.int32, (1, TPB), 1)
    valid = (ip * TPB + lanes) < P
    only_l = jnp.where(valid, ml_ref[...], 0.0)
    only_r = jnp.where(valid, mr_ref[...], 0.0)

    tl = loc_ref[...]
    tls = _roll(locsh_ref[...])
    tli = loci_ref[...]
    se_l = jnp.where(valid, jnp.sum((tli - tl) ** 2, axis=1), 0.0)
    se_r = jnp.where(valid, jnp.sum((tli - tls) ** 2, axis=1), 0.0)

    @pl.when(ip == 0)
    def _init():
        acc_ref[...] = jnp.zeros((2, B, TPB), jnp.float32)

    acc_ref[0] += se_l * only_l
    acc_ref[1] += se_r * only_r

    @pl.when(ip == NPB - 1)
    def _fin():
        for j in range(2):
            out_ref[j] = jnp.sum(acc_ref[j])


@jax.jit
def _isd_loss(lam, conf, conf_shuffle, conf_interpolation, loc, loc_shuffle,
              loc_interpolation):
    cspec = pl.BlockSpec((C, B, TP), lambda ip: (0, 0, ip))
    mspec = pl.BlockSpec((B, TP), lambda ip: (0, ip))

    sums6, mask_l, mask_r = pl.pallas_call(
        _conf_kernel,
        grid=(NP,),
        in_specs=[pl.BlockSpec(memory_space=pltpu.SMEM), cspec, cspec, cspec],
        out_specs=[pl.BlockSpec(memory_space=pltpu.SMEM), mspec, mspec],
        out_shape=[jax.ShapeDtypeStruct((6,), jnp.float32),
                   jax.ShapeDtypeStruct((B, P), jnp.float32),
                   jax.ShapeDtypeStruct((B, P), jnp.float32)],
        scratch_shapes=[pltpu.VMEM((6, B, TP), jnp.float32)],
        compiler_params=pltpu.CompilerParams(
            dimension_semantics=("arbitrary",),
        ),
    )(lam.reshape(1).astype(jnp.float32),
      jnp.transpose(conf, (2, 0, 1)), jnp.transpose(conf_shuffle, (2, 0, 1)),
      jnp.transpose(conf_interpolation, (2, 0, 1)))

    mspec_b = pl.BlockSpec((B, TPB), lambda ip: (0, ip))
    lspec = pl.BlockSpec((B, 4, TPB), lambda ip: (0, 0, ip))
    sums2 = pl.pallas_call(
        _loc_kernel,
        grid=(NPB,),
        in_specs=[mspec_b, mspec_b, lspec, lspec, lspec],
        out_specs=pl.BlockSpec(memory_space=pltpu.SMEM),
        out_shape=jax.ShapeDtypeStruct((2,), jnp.float32),
        scratch_shapes=[pltpu.VMEM((2, B, TPB), jnp.float32)],
        compiler_params=pltpu.CompilerParams(
            dimension_semantics=("arbitrary",),
        ),
    )(mask_l, mask_r,
      jnp.transpose(loc, (0, 2, 1)), jnp.transpose(loc_shuffle, (0, 2, 1)),
      jnp.transpose(loc_interpolation, (0, 2, 1)))

    def mmean(s, c):
        return jnp.where(c > 0, s / jnp.maximum(c, 1.0), 0.0)

    interp_loss = mmean(sums6[0], sums6[1]) / 2.0
    fixmatch = (mmean(sums6[2], sums6[3]) + mmean(sums2[0], sums6[3]) / 4.0
                + mmean(sums6[4], sums6[5]) + mmean(sums2[1], sums6[5]) / 4.0)
    return interp_loss, fixmatch


def kernel(lam, conf, conf_flip, loc, loc_flip, conf_shuffle,
           conf_interpolation, loc_shuffle, loc_interpolation):
    del conf_flip, loc_flip
    return _isd_loss(lam, conf, conf_shuffle, conf_interpolation, loc,
                     loc_shuffle, loc_interpolation)


# speedup vs baseline: 1.1172x; 1.1172x over previous
import jax
import jax.numpy as jnp
from jax.experimental import pallas as pl
from jax.experimental.pallas import tpu as pltpu

B, P, C = 32, 8732, 21
TP = 384
NP = 23
HALF = B // 2


def _roll(x):
    return jnp.concatenate([x[HALF:], x[:HALF]], axis=0)


def _isd_kernel(lam_ref, conf_ref, confsh_ref, confi_ref, loc_ref, locsh_ref,
                loci_ref, out_ref, acc_ref):
    ip = pl.program_id(0)
    lam = lam_ref[0]
    q = 1.0 - lam

    lanes = jax.lax.broadcasted_iota(jnp.int32, (1, TP), 1)
    valid = (ip * TP + lanes) < P
    validf = valid.astype(jnp.float32)

    tc0 = conf_ref[0]
    tt0 = _roll(confsh_ref[0])

    s_ab = jnp.zeros((B, TP), jnp.float32)
    s_l = jnp.zeros((B, TP), jnp.float32)
    s_r = jnp.zeros((B, TP), jnp.float32)
    m_l = tc0
    m_r = tt0
    for c in range(C):
        tc_c = tc0 if c == 0 else conf_ref[c]
        tt_c = tt0 if c == 0 else _roll(confsh_ref[c])
        if c > 0:
            m_l = jnp.maximum(m_l, tc_c)
            m_r = jnp.maximum(m_r, tt_c)
        interp = confi_ref[c] + 1e-07
        mixed = lam * tc_c + q * tt_c + 1e-07
        log_i = jnp.log(interp)
        d = log_i - jnp.log(mixed)
        s_ab += (interp - mixed) * d
        ce = tc_c + 1e-07
        cte = tt_c + 1e-07
        s_l += ce * (jnp.log(ce) - log_i)
        s_r += cte * (jnp.log(cte) - log_i)

    lmf = (m_l > tc0).astype(jnp.float32) * validf
    rmf = (m_r > tt0).astype(jnp.float32) * validf
    inter = lmf * rmf
    only_l = lmf - inter
    only_r = rmf - inter

    s_ab = jnp.where(valid, s_ab, 0.0)
    s_l = jnp.where(valid, s_l, 0.0)
    s_r = jnp.where(valid, s_r, 0.0)

    tl = loc_ref[...]
    tls = _roll(locsh_ref[...])
    tli = loci_ref[...]
    se_l = jnp.where(valid, jnp.sum((tli - tl) ** 2, axis=1), 0.0)
    se_r = jnp.where(valid, jnp.sum((tli - tls) ** 2, axis=1), 0.0)

    @pl.when(ip == 0)
    def _init():
        acc_ref[...] = jnp.zeros((8, B, TP), jnp.float32)

    acc_ref[0] += s_ab * inter
    acc_ref[1] += inter
    acc_ref[2] += s_l * only_l
    acc_ref[3] += only_l
    acc_ref[4] += s_r * only_r
    acc_ref[5] += only_r
    acc_ref[6] += se_l * only_l
    acc_ref[7] += se_r * only_r

    @pl.when(ip == NP - 1)
    def _fin():
        for j in range(8):
            out_ref[j] = jnp.sum(acc_ref[j])


@jax.jit
def _isd_loss(lam, conf, conf_shuffle, conf_interpolation, loc, loc_shuffle,
              loc_interpolation):
    cspec = pl.BlockSpec((C, B, TP), lambda ip: (0, 0, ip))
    lspec = pl.BlockSpec((B, 4, TP), lambda ip: (0, 0, ip))

    sums = pl.pallas_call(
        _isd_kernel,
        grid=(NP,),
        in_specs=[
            pl.BlockSpec(memory_space=pltpu.SMEM),
            cspec, cspec, cspec, lspec, lspec, lspec,
        ],
        out_specs=pl.BlockSpec(memory_space=pltpu.SMEM),
        out_shape=jax.ShapeDtypeStruct((8,), jnp.float32),
        scratch_shapes=[pltpu.VMEM((8, B, TP), jnp.float32)],
        compiler_params=pltpu.CompilerParams(
            dimension_semantics=("arbitrary",),
        ),
    )(lam.reshape(1).astype(jnp.float32),
      jnp.transpose(conf, (2, 0, 1)), jnp.transpose(conf_shuffle, (2, 0, 1)),
      jnp.transpose(conf_interpolation, (2, 0, 1)),
      jnp.transpose(loc, (0, 2, 1)), jnp.transpose(loc_shuffle, (0, 2, 1)),
      jnp.transpose(loc_interpolation, (0, 2, 1)))

    def mmean(s, c):
        return jnp.where(c > 0, s / jnp.maximum(c, 1.0), 0.0)

    interp_loss = mmean(sums[0], sums[1]) / 2.0
    fixmatch = (mmean(sums[2], sums[3]) + mmean(sums[6], sums[3]) / 4.0
                + mmean(sums[4], sums[5]) + mmean(sums[7], sums[5]) / 4.0)
    return interp_loss, fixmatch


def kernel(lam, conf, conf_flip, loc, loc_flip, conf_shuffle,
           conf_interpolation, loc_shuffle, loc_interpolation):
    del conf_flip, loc_flip
    return _isd_loss(lam, conf, conf_shuffle, conf_interpolation, loc,
                     loc_shuffle, loc_interpolation)
